# baseline (speedup 1.0000x reference)
import functools

import jax
import jax.numpy as jnp
from jax import lax
from jax.experimental import pallas as pl
from jax.experimental.pallas import tpu as pltpu
from jax.experimental.pallas import tpu_sc as plsc

LANES = 16
CHUNK = 32
NBUF = 3


def _build_sc_call(B, L, E):
    RIN = L * E
    ROUT = (2 * L + 1) * E
    info = plsc.get_sparse_core_info()
    NC, NS = info.num_cores, info.num_subcores
    NW = NC * NS
    HALF = NW // 2
    NCHUNK = RIN // CHUNK
    assert RIN % CHUNK == 0 and NCHUNK % HALF == 0 and E % LANES == 0
    CPW = NCHUNK // HALF
    NVC = B // LANES
    PER = 2 * CHUNK
    assert PER == E and E % CHUNK == 0

    def body(x1_hbm, x2_hbm, pat_hbm, out_hbm,
             b0, b1, b2, pat, s0, s1, s2, t0, t1, t2):
        wid = lax.axis_index("s") * NC + lax.axis_index("c")
        bufs = (b0, b1, b2)
        sin = (s0, s1, s2)
        sout = (t0, t1, t2)

        pltpu.sync_copy(pat_hbm, pat)

        def run_stream(src, outoff, patoff, widx):
            def rs_of(t):
                return (widx * CPW + t) * CHUNK

            def issue_in(t, k):
                pltpu.async_copy(src.at[pl.ds(rs_of(t), CHUNK)],
                                 bufs[k], sin[k])

            def wait_in(k):
                pltpu.make_async_copy(src.at[pl.ds(0, CHUNK)],
                                      bufs[k], sin[k]).wait()

            def issue_out(t, k):
                pltpu.async_copy(bufs[k],
                                 out_hbm.at[pl.ds(outoff + rs_of(t), CHUNK)],
                                 sout[k])

            def wait_out(k):
                pltpu.make_async_copy(bufs[k],
                                      out_hbm.at[pl.ds(0, CHUNK)],
                                      sout[k]).wait()

            def compute(t, k):
                w = bufs[k]
                pb = patoff + (rs_of(t) & (PER - 1)) * LANES
                for half in range(CHUNK // LANES):
                    hb = pb + half * LANES * LANES
                    splats = [pat[pl.ds(hb + j * LANES, LANES)]
                              for j in range(LANES)]

                    @plsc.parallel_loop(0, NVC, step=1, unroll=2)
                    def _(v):
                        s = pl.ds(v * LANES, LANES)
                        for j in range(LANES):
                            r = half * LANES + j
                            w[r, s] = w[r, s] + splats[j]

            issue_in(0, 0)
            issue_in(1, 1)

            def loop_body(g, carry):
                for k in range(NBUF):
                    t = g * NBUF + k

                    @pl.when(t < CPW)
                    def _():
                        wait_in(k)
                        compute(t, k)
                        issue_out(t, k)

                    @pl.when((t >= 1) & (t + 2 < CPW))
                    def _():
                        wait_out((k + 2) % NBUF)

                    @pl.when(t + 2 < CPW)
                    def _():
                        issue_in(t + 2, (k + 2) % NBUF)
                return carry

            lax.fori_loop(0, (CPW + NBUF - 1) // NBUF, loop_body, 0)
            wait_out((CPW - 3) % NBUF)
            wait_out((CPW - 2) % NBUF)
            wait_out((CPW - 1) % NBUF)

        @pl.when(wid < HALF)
        def _():
            run_stream(x1_hbm, E, 0, wid)

        @pl.when(wid >= HALF)
        def _():
            run_stream(x2_hbm, E + RIN, E * LANES, wid - HALF)

        @pl.when(wid < E // CHUNK)
        def _():
            base = wid * CHUNK

            @plsc.parallel_loop(0, NVC, step=1, unroll=2)
            def _(v):
                s = pl.ds(v * LANES, LANES)
                for j in range(CHUNK):
                    b0[j, s] = pat[pl.ds((2 * E + base + j) * LANES, LANES)]

            pltpu.sync_copy(b0, out_hbm.at[pl.ds(base, CHUNK)])

    mesh = plsc.VectorSubcoreMesh(core_axis_name="c", subcore_axis_name="s")
    return pl.kernel(
        body,
        mesh=mesh,
        out_type=jax.ShapeDtypeStruct((ROUT, B), jnp.float32),
        scratch_types=[
            pltpu.VMEM((CHUNK, B), jnp.float32),
            pltpu.VMEM((CHUNK, B), jnp.float32),
            pltpu.VMEM((CHUNK, B), jnp.float32),
            pltpu.VMEM((3 * E * LANES,), jnp.float32),
            pltpu.SemaphoreType.DMA,
            pltpu.SemaphoreType.DMA,
            pltpu.SemaphoreType.DMA,
            pltpu.SemaphoreType.DMA,
            pltpu.SemaphoreType.DMA,
            pltpu.SemaphoreType.DMA,
        ],
    )


def kernel(x1, x2, cls_token, type_embeddings):
    B, L, E = x1.shape
    call = _build_sc_call(B, L, E)
    x1v = x1.transpose(1, 2, 0).reshape(L * E, B)
    x2v = x2.transpose(1, 2, 0).reshape(L * E, B)
    scal = jnp.concatenate(
        [type_embeddings.reshape(2 * E), cls_token.reshape(E)])
    pat = jnp.repeat(scal, LANES)
    outv = call(x1v, x2v, pat)
    return outv.reshape(2 * L + 1, E, B).transpose(2, 0, 1)


# speedup vs baseline: 1.0907x; 1.0907x over previous
import functools

import jax
import jax.numpy as jnp
from jax import lax
from jax.experimental import pallas as pl
from jax.experimental.pallas import tpu as pltpu
from jax.experimental.pallas import tpu_sc as plsc

LANES = 16
CHUNK = 16


def _build_sc_call(B, L, E):
    RIN = L * E
    ROUT = (2 * L + 1) * E
    info = plsc.get_sparse_core_info()
    NC, NS = info.num_cores, info.num_subcores
    NW = NC * NS
    NCHUNK = RIN // CHUNK
    assert RIN % CHUNK == 0 and NCHUNK % NW == 0 and E % LANES == 0
    CPW = NCHUNK // NW
    NVC = B // LANES

    def body(x1_hbm, x2_hbm, pat_hbm, out_hbm,
             ia, ib, oa, ob, pat,
             sina, sinb, souta, soutb):
        wid = lax.axis_index("s") * NC + lax.axis_index("c")
        ibuf = (ia, ib)
        obuf = (oa, ob)
        sin = (sina, sinb)
        sout = (souta, soutb)
        srcs = (x1_hbm, x2_hbm)
        outoff = (E, E + RIN)
        patbase = (0, E * LANES)

        pltpu.sync_copy(pat_hbm, pat)

        def rs_of(c):
            return (wid * CPW + c) * CHUNK

        def issue_in(c, p):
            pltpu.async_copy(srcs[p].at[pl.ds(rs_of(c), CHUNK)],
                             ibuf[p], sin[p])

        def wait_in(p):
            pltpu.make_async_copy(srcs[p].at[pl.ds(0, CHUNK)],
                                  ibuf[p], sin[p]).wait()

        def issue_out(c, p):
            pltpu.async_copy(obuf[p],
                             out_hbm.at[pl.ds(outoff[p] + rs_of(c), CHUNK)],
                             sout[p])

        def wait_out(p):
            pltpu.make_async_copy(obuf[p],
                                  out_hbm.at[pl.ds(0, CHUNK)],
                                  sout[p]).wait()

        def compute(c, p):
            r, w = ibuf[p], obuf[p]
            pb = patbase[p] + (rs_of(c) & (E - 1)) * LANES
            splats = [pat[pl.ds(pb + j * LANES, LANES)] for j in range(CHUNK)]

            @plsc.parallel_loop(0, NVC, step=1, unroll=4)
            def _(v):
                s = pl.ds(v * LANES, LANES)
                for j in range(CHUNK):
                    w[j, s] = r[j, s] + splats[j]

        issue_in(0, 0)
        issue_in(0, 1)

        for p in range(2):
            wait_in(p)
            compute(0, p)
            issue_out(0, p)
            issue_in(1, p)

        def loop_body(c, carry):
            for p in range(2):
                wait_in(p)
                wait_out(p)
                compute(c, p)
                issue_out(c, p)
                issue_in(c + 1, p)
            return carry

        lax.fori_loop(1, CPW - 1, loop_body, 0)

        for p in range(2):
            wait_in(p)
            wait_out(p)
            compute(CPW - 1, p)
            issue_out(CPW - 1, p)
        wait_out(0)
        wait_out(1)

        @pl.when(wid < E // CHUNK)
        def _():
            base = wid * CHUNK

            @plsc.parallel_loop(0, NVC, step=1, unroll=4)
            def _(v):
                s = pl.ds(v * LANES, LANES)
                for j in range(CHUNK):
                    oa[j, s] = pat[pl.ds((2 * E + base + j) * LANES, LANES)]

            pltpu.sync_copy(oa, out_hbm.at[pl.ds(base, CHUNK)])

    mesh = plsc.VectorSubcoreMesh(core_axis_name="c", subcore_axis_name="s")
    return pl.kernel(
        body,
        mesh=mesh,
        out_type=jax.ShapeDtypeStruct((ROUT, B), jnp.float32),
        scratch_types=[
            pltpu.VMEM((CHUNK, B), jnp.float32),
            pltpu.VMEM((CHUNK, B), jnp.float32),
            pltpu.VMEM((CHUNK, B), jnp.float32),
            pltpu.VMEM((CHUNK, B), jnp.float32),
            pltpu.VMEM((3 * E * LANES,), jnp.float32),
            pltpu.SemaphoreType.DMA,
            pltpu.SemaphoreType.DMA,
            pltpu.SemaphoreType.DMA,
            pltpu.SemaphoreType.DMA,
        ],
    )


def kernel(x1, x2, cls_token, type_embeddings):
    B, L, E = x1.shape
    call = _build_sc_call(B, L, E)
    x1v = x1.transpose(1, 2, 0).reshape(L * E, B)
    x2v = x2.transpose(1, 2, 0).reshape(L * E, B)
    scal = jnp.concatenate(
        [type_embeddings.reshape(2 * E), cls_token.reshape(E)])
    pat = jnp.repeat(scal, LANES)
    outv = call(x1v, x2v, pat)
    return outv.reshape(2 * L + 1, E, B).transpose(2, 0, 1)


# speedup vs baseline: 1.0996x; 1.0081x over previous
import functools

import jax
import jax.numpy as jnp
from jax import lax
from jax.experimental import pallas as pl
from jax.experimental.pallas import tpu as pltpu
from jax.experimental.pallas import tpu_sc as plsc

LANES = 16
CHUNK = 16


def _build_sc_call(B, L, E):
    RIN = L * E
    ROUT = (2 * L + 1) * E
    info = plsc.get_sparse_core_info()
    NC, NS = info.num_cores, info.num_subcores
    NW = NC * NS
    NCHUNK = RIN // CHUNK
    assert RIN % CHUNK == 0 and NCHUNK % NW == 0 and E % LANES == 0
    CPW = NCHUNK // NW
    NVC = B // LANES

    def body(x1_hbm, x2_hbm, pat_hbm, out_hbm,
             ia, ib, oa, ob, pat,
             sina, sinb, souta, soutb):
        wid = lax.axis_index("s") * NC + lax.axis_index("c")
        ibuf = (ia, ib)
        obuf = (oa, ob)
        sin = (sina, sinb)
        sout = (souta, soutb)
        srcs = (x1_hbm, x2_hbm)
        outoff = (E, E + RIN)
        patbase = (0, E * LANES)

        pltpu.sync_copy(pat_hbm, pat)

        def rs_of(c):
            return (wid * CPW + c) * CHUNK

        def issue_in(c, p):
            pltpu.async_copy(srcs[p].at[pl.ds(rs_of(c), CHUNK)],
                             ibuf[p], sin[p])

        def wait_in(p):
            pltpu.make_async_copy(srcs[p].at[pl.ds(0, CHUNK)],
                                  ibuf[p], sin[p]).wait()

        def issue_out(c, p):
            pltpu.async_copy(obuf[p],
                             out_hbm.at[pl.ds(outoff[p] + rs_of(c), CHUNK)],
                             sout[p])

        def wait_out(p):
            pltpu.make_async_copy(obuf[p],
                                  out_hbm.at[pl.ds(0, CHUNK)],
                                  sout[p]).wait()

        def compute(c, p):
            r, w = ibuf[p], obuf[p]
            pb = patbase[p] + (rs_of(c) & (E - 1)) * LANES
            splats = [pat[pl.ds(pb + j * LANES, LANES)] for j in range(CHUNK)]

            @plsc.parallel_loop(0, NVC, step=1, unroll=2)
            def _(v):
                s = pl.ds(v * LANES, LANES)
                for j in range(CHUNK):
                    w[j, s] = r[j, s] + splats[j]

        issue_in(0, 0)
        issue_in(0, 1)

        for p in range(2):
            wait_in(p)
            compute(0, p)
            issue_out(0, p)
            issue_in(1, p)

        def loop_body(c, carry):
            for p in range(2):
                wait_in(p)
                wait_out(p)
                compute(c, p)
                issue_out(c, p)
                issue_in(c + 1, p)
            return carry

        lax.fori_loop(1, CPW - 1, loop_body, 0)

        for p in range(2):
            wait_in(p)
            wait_out(p)
            compute(CPW - 1, p)
            issue_out(CPW - 1, p)
        wait_out(0)
        wait_out(1)

        @pl.when(wid < E // CHUNK)
        def _():
            base = wid * CHUNK

            @plsc.parallel_loop(0, NVC, step=1, unroll=2)
            def _(v):
                s = pl.ds(v * LANES, LANES)
                for j in range(CHUNK):
                    oa[j, s] = pat[pl.ds((2 * E + base + j) * LANES, LANES)]

            pltpu.sync_copy(oa, out_hbm.at[pl.ds(base, CHUNK)])

    mesh = plsc.VectorSubcoreMesh(core_axis_name="c", subcore_axis_name="s")
    return pl.kernel(
        body,
        mesh=mesh,
        out_type=jax.ShapeDtypeStruct((ROUT, B), jnp.float32),
        scratch_types=[
            pltpu.VMEM((CHUNK, B), jnp.float32),
            pltpu.VMEM((CHUNK, B), jnp.float32),
            pltpu.VMEM((CHUNK, B), jnp.float32),
            pltpu.VMEM((CHUNK, B), jnp.float32),
            pltpu.VMEM((3 * E * LANES,), jnp.float32),
            pltpu.SemaphoreType.DMA,
            pltpu.SemaphoreType.DMA,
            pltpu.SemaphoreType.DMA,
            pltpu.SemaphoreType.DMA,
        ],
    )


def kernel(x1, x2, cls_token, type_embeddings):
    B, L, E = x1.shape
    call = _build_sc_call(B, L, E)
    x1v = x1.transpose(1, 2, 0).reshape(L * E, B)
    x2v = x2.transpose(1, 2, 0).reshape(L * E, B)
    scal = jnp.concatenate(
        [type_embeddings.reshape(2 * E), cls_token.reshape(E)])
    pat = jnp.repeat(scal, LANES)
    outv = call(x1v, x2v, pat)
    return outv.reshape(2 * L + 1, E, B).transpose(2, 0, 1)


# speedup vs baseline: 1.1276x; 1.0255x over previous
import functools

import jax
import jax.numpy as jnp
from jax import lax
from jax.experimental import pallas as pl
from jax.experimental.pallas import tpu as pltpu
from jax.experimental.pallas import tpu_sc as plsc

LANES = 16
CHUNK = 16


def _build_sc_call(B, L, E):
    RIN = L * E
    ROUT = (2 * L + 1) * E
    info = plsc.get_sparse_core_info()
    NC, NS = info.num_cores, info.num_subcores
    NW = NC * NS
    NCHUNK = RIN // CHUNK
    assert RIN % CHUNK == 0 and NCHUNK % NW == 0 and E % LANES == 0
    CPW = NCHUNK // NW
    NVC = B // LANES

    def body(x1_hbm, x2_hbm, pat_hbm, out_hbm,
             ia, ib, oa, ob, pat,
             sina, sinb, souta, soutb):
        wid = lax.axis_index("s") * NC + lax.axis_index("c")
        ibuf = (ia, ib)
        obuf = (oa, ob)
        sin = (sina, sinb)
        sout = (souta, soutb)
        srcs = (x1_hbm, x2_hbm)
        outoff = (E, E + RIN)
        patbase = (0, E * LANES)

        pltpu.sync_copy(pat_hbm, pat)

        def rs_of(c):
            return (wid * CPW + c) * CHUNK

        def issue_in(c, p):
            pltpu.async_copy(srcs[p].at[pl.ds(rs_of(c), CHUNK)],
                             ibuf[p], sin[p])

        def wait_in(p):
            pltpu.make_async_copy(srcs[p].at[pl.ds(0, CHUNK)],
                                  ibuf[p], sin[p]).wait()

        def issue_out(c, p):
            pltpu.async_copy(obuf[p],
                             out_hbm.at[pl.ds(outoff[p] + rs_of(c), CHUNK)],
                             sout[p])

        def wait_out(p):
            pltpu.make_async_copy(obuf[p],
                                  out_hbm.at[pl.ds(0, CHUNK)],
                                  sout[p]).wait()

        def compute(c, p):
            r, w = ibuf[p], obuf[p]
            pb = patbase[p] + (rs_of(c) & (E - 1)) * LANES
            splats = [pat[pl.ds(pb + j * LANES, LANES)] for j in range(CHUNK)]

            @plsc.parallel_loop(0, NVC, step=1, unroll=2)
            def _(v):
                s = pl.ds(v * LANES, LANES)
                for j in range(CHUNK):
                    w[j, s] = r[j, s] + splats[j]

        issue_in(0, 0)
        issue_in(0, 1)

        def loop_body(c, carry):
            for p in range(2):
                wait_in(p)

                @pl.when(c > 0)
                def _():
                    wait_out(p)

                compute(c, p)
                issue_out(c, p)

                @pl.when(c < CPW - 1)
                def _():
                    issue_in(c + 1, p)
            return carry

        lax.fori_loop(0, CPW, loop_body, 0)
        wait_out(0)
        wait_out(1)

        @pl.when(wid < E // CHUNK)
        def _():
            base = wid * CHUNK

            @plsc.parallel_loop(0, NVC, step=1, unroll=2)
            def _(v):
                s = pl.ds(v * LANES, LANES)
                for j in range(CHUNK):
                    oa[j, s] = pat[pl.ds((2 * E + base + j) * LANES, LANES)]

            pltpu.sync_copy(oa, out_hbm.at[pl.ds(base, CHUNK)])

    mesh = plsc.VectorSubcoreMesh(core_axis_name="c", subcore_axis_name="s")
    return pl.kernel(
        body,
        mesh=mesh,
        out_type=jax.ShapeDtypeStruct((ROUT, B), jnp.float32),
        scratch_types=[
            pltpu.VMEM((CHUNK, B), jnp.float32),
            pltpu.VMEM((CHUNK, B), jnp.float32),
            pltpu.VMEM((CHUNK, B), jnp.float32),
            pltpu.VMEM((CHUNK, B), jnp.float32),
            pltpu.VMEM((3 * E * LANES,), jnp.float32),
            pltpu.SemaphoreType.DMA,
            pltpu.SemaphoreType.DMA,
            pltpu.SemaphoreType.DMA,
            pltpu.SemaphoreType.DMA,
        ],
    )


def kernel(x1, x2, cls_token, type_embeddings):
    B, L, E = x1.shape
    call = _build_sc_call(B, L, E)
    x1v = x1.transpose(1, 2, 0).reshape(L * E, B)
    x2v = x2.transpose(1, 2, 0).reshape(L * E, B)
    scal = jnp.concatenate(
        [type_embeddings.reshape(2 * E), cls_token.reshape(E)])
    pat = jnp.repeat(scal, LANES)
    outv = call(x1v, x2v, pat)
    return outv.reshape(2 * L + 1, E, B).transpose(2, 0, 1)
